# baseline (speedup 1.0000x reference)
import jax
import jax.numpy as jnp
from jax import lax
from jax.experimental import pallas as pl
from jax.experimental.pallas import tpu as pltpu
from jax.experimental.pallas import tpu_sc as plsc

N = 10000
E = 320000
D = 128
H = 128
O = 64

NC = 2
NS = 16
NW = NC * NS
B = 128
CH = 8

EP = ((E + NW * B * CH - 1) // (NW * B * CH)) * (NW * B * CH)
NROW = EP // B
C0 = 10
C1 = 10
ROWS0 = C0 * CH
ROWS1 = C1 * CH

NP = 10112
NPT = NP // NS
NPAD = NP
NR = 80
NCNT = NR * B

_mesh = plsc.VectorSubcoreMesh(
    core_axis_name="c", subcore_axis_name="s", num_cores=NC, num_subcores=NS
)


def _agg_pipeline(feat, src2d, dst2d, agg_sh, c, s,
                  idx_s, idx_d, rows0, rows1, gs0, gs1, ss0, ss1,
                  cnt_v=None):
    bufs = (rows0, rows1)
    gsems = (gs0, gs1)
    ssems = (ss0, ss1)
    ones16 = jnp.ones((16,), jnp.float32)
    base = jnp.where(c == 0, s * ROWS0, NS * ROWS0 + s * ROWS1)
    nch = jnp.where(c == 0, C0, C1)

    def step(j, carry):
        row0 = base + j * CH
        pltpu.sync_copy(src2d.at[pl.ds(row0, CH)], idx_s)
        pltpu.sync_copy(dst2d.at[pl.ds(row0, CH)], idx_d)
        g = {}
        sc = {}
        g[0] = pltpu.async_copy(feat.at[idx_s.at[0]], bufs[0], gsems[0])
        for k in range(CH):
            if k + 1 < CH:
                if k >= 1:
                    sc[k - 1].wait()
                g[k + 1] = pltpu.async_copy(
                    feat.at[idx_s.at[k + 1]], bufs[(k + 1) % 2],
                    gsems[(k + 1) % 2])
            if cnt_v is not None:
                for gi in range(B // 16):
                    idxv = idx_d[k, pl.ds(gi * 16, 16)]
                    plsc.addupdate_scatter(cnt_v, [idxv], ones16)
            g[k].wait()
            sc[k] = pltpu.async_copy(bufs[k % 2], agg_sh.at[idx_d.at[k]],
                                     ssems[k % 2], add=True)
        sc[CH - 2].wait()
        sc[CH - 1].wait()
        return carry

    lax.fori_loop(0, nch, step, 0)


def _sc_agg1_body(feat, src2d, dst2d, zero_agg, zero_cnt,
                  agg_out, cnt_out,
                  idx_s, idx_d, rows0, rows1, cnt_v, agg_sh,
                  gs0, gs1, ss0, ss1):
    c = lax.axis_index("c")
    s = lax.axis_index("s")
    tid = c * NS + s
    r0 = s * NPT
    pltpu.sync_copy(zero_agg.at[pl.ds(r0, NPT)], agg_sh.at[pl.ds(r0, NPT)])
    pltpu.sync_copy(zero_cnt, cnt_v)
    plsc.subcore_barrier()
    _agg_pipeline(feat, src2d, dst2d, agg_sh, c, s,
                  idx_s, idx_d, rows0, rows1, gs0, gs1, ss0, ss1, cnt_v)
    plsc.subcore_barrier()
    pltpu.sync_copy(agg_sh.at[pl.ds(r0, NPT)], agg_out.at[c, pl.ds(r0, NPT)])
    pltpu.sync_copy(cnt_v, cnt_out.at[pl.ds(tid * NCNT, NCNT)])


def _sc_agg2_body(feat, src2d, dst2d, zero_agg,
                  agg_out,
                  idx_s, idx_d, rows0, rows1, agg_sh,
                  gs0, gs1, ss0, ss1):
    c = lax.axis_index("c")
    s = lax.axis_index("s")
    tid = c * NS + s
    r0 = s * NPT
    pltpu.sync_copy(zero_agg.at[pl.ds(r0, NPT)], agg_sh.at[pl.ds(r0, NPT)])
    plsc.subcore_barrier()
    _agg_pipeline(feat, src2d, dst2d, agg_sh, c, s,
                  idx_s, idx_d, rows0, rows1, gs0, gs1, ss0, ss1)
    plsc.subcore_barrier()
    pltpu.sync_copy(agg_sh.at[pl.ds(r0, NPT)], agg_out.at[c, pl.ds(r0, NPT)])


_sc_agg1 = pl.kernel(
    _sc_agg1_body,
    out_type=[
        jax.ShapeDtypeStruct((NC, NP, D), jnp.float32),
        jax.ShapeDtypeStruct((NW * NCNT,), jnp.float32),
    ],
    mesh=_mesh,
    compiler_params=pltpu.CompilerParams(needs_layout_passes=False),
    scratch_types=[
        pltpu.VMEM((CH, B), jnp.int32),
        pltpu.VMEM((CH, B), jnp.int32),
        pltpu.VMEM((B, D), jnp.float32),
        pltpu.VMEM((B, D), jnp.float32),
        pltpu.VMEM((NCNT,), jnp.float32),
        pltpu.VMEM_SHARED((NP, D), jnp.float32),
        pltpu.SemaphoreType.DMA,
        pltpu.SemaphoreType.DMA,
        pltpu.SemaphoreType.DMA,
        pltpu.SemaphoreType.DMA,
    ],
)

_sc_agg2 = pl.kernel(
    _sc_agg2_body,
    out_type=jax.ShapeDtypeStruct((NC, NP, H), jnp.float32),
    mesh=_mesh,
    scratch_types=[
        pltpu.VMEM((CH, B), jnp.int32),
        pltpu.VMEM((CH, B), jnp.int32),
        pltpu.VMEM((B, H), jnp.float32),
        pltpu.VMEM((B, H), jnp.float32),
        pltpu.VMEM_SHARED((NP, H), jnp.float32),
        pltpu.SemaphoreType.DMA,
        pltpu.SemaphoreType.DMA,
        pltpu.SemaphoreType.DMA,
        pltpu.SemaphoreType.DMA,
    ],
)

_BR = 128


PAD_E = EP - E


def _cnt_col(cntp):
    cnt = jnp.sum(cntp[:, 0, 0, :], axis=0)
    rowid = pl.program_id(0) * B + lax.iota(jnp.int32, B)
    cnt = cnt - jnp.where(rowid < PAD_E, 1.0, 0.0)
    return jnp.maximum(cnt, 1.0).reshape(B, 1)


def _tc1_body(aggp, cntp, x, wl, bl, wr, out):
    agg = aggp[0] + aggp[1]
    mean = agg / _cnt_col(cntp)
    r = lax.dot_general(mean, wl[...], (((1,), (1,)), ((), ())),
                        preferred_element_type=jnp.float32)
    r = r + bl[...]
    r = r + lax.dot_general(x[...], wr[...], (((1,), (1,)), ((), ())),
                            preferred_element_type=jnp.float32)
    rowid = pl.program_id(0) * B + lax.broadcasted_iota(jnp.int32, (B, 1), 0)[:, 0]
    keep = (rowid < N).reshape(B, 1)
    out[...] = jnp.where(keep, jnp.maximum(r, 0.0), 0.0)


def _tc2_body(aggp, cntp, h, wl, bl, wr, wlin, blin, out):
    agg = aggp[0] + aggp[1]
    mean = agg / _cnt_col(cntp)
    r = lax.dot_general(mean, wl[...], (((1,), (1,)), ((), ())),
                        preferred_element_type=jnp.float32)
    r = r + bl[...]
    r = r + lax.dot_general(h[...], wr[...], (((1,), (1,)), ((), ())),
                            preferred_element_type=jnp.float32)
    logits = lax.dot_general(r, wlin[...], (((1,), (1,)), ((), ())),
                             preferred_element_type=jnp.float32)
    logits = logits + blin[...]
    m = jnp.max(logits, axis=1, keepdims=True)
    lse = jnp.log(jnp.sum(jnp.exp(logits - m), axis=1, keepdims=True)) + m
    out[...] = logits - lse


def _tc1(agg_parts, cnt_parts, xp, W_l1, b_l1, W_r1):
    return pl.pallas_call(
        _tc1_body,
        grid=(NPAD // _BR,),
        in_specs=[
            pl.BlockSpec((NC, _BR, D), lambda i: (0, i, 0)),
            pl.BlockSpec((NW, 1, 1, B), lambda i: (0, i, 0, 0)),
            pl.BlockSpec((_BR, D), lambda i: (i, 0)),
            pl.BlockSpec((H, D), lambda i: (0, 0)),
            pl.BlockSpec((1, H), lambda i: (0, 0)),
            pl.BlockSpec((H, D), lambda i: (0, 0)),
        ],
        out_specs=pl.BlockSpec((_BR, H), lambda i: (i, 0)),
        out_shape=jax.ShapeDtypeStruct((NPAD, H), jnp.float32),
    )(agg_parts, cnt_parts, xp, W_l1, b_l1.reshape(1, H), W_r1)


def _tc2(agg_parts, cnt_parts, h, W_l2, b_l2, W_r2, W_lin, b_lin):
    return pl.pallas_call(
        _tc2_body,
        grid=(NPAD // _BR,),
        in_specs=[
            pl.BlockSpec((NC, _BR, H), lambda i: (0, i, 0)),
            pl.BlockSpec((NW, 1, 1, B), lambda i: (0, i, 0, 0)),
            pl.BlockSpec((_BR, H), lambda i: (i, 0)),
            pl.BlockSpec((H, H), lambda i: (0, 0)),
            pl.BlockSpec((1, H), lambda i: (0, 0)),
            pl.BlockSpec((H, H), lambda i: (0, 0)),
            pl.BlockSpec((O, H), lambda i: (0, 0)),
            pl.BlockSpec((1, O), lambda i: (0, 0)),
        ],
        out_specs=pl.BlockSpec((_BR, O), lambda i: (i, 0)),
        out_shape=jax.ShapeDtypeStruct((NPAD, O), jnp.float32),
    )(agg_parts, cnt_parts, h, W_l2, b_l2.reshape(1, H), W_r2,
      W_lin, b_lin.reshape(1, O))


def kernel(x, edge_index, W_l1, b_l1, W_r1, W_l2, b_l2, W_r2, W_lin, b_lin):
    src = edge_index[0]
    dst = edge_index[1]
    pad = EP - E
    src2d = jnp.concatenate(
        [src, jnp.full((pad,), N, jnp.int32)]).reshape(NROW, B)
    pad_dst = jnp.arange(pad, dtype=jnp.int32)
    dst2d = jnp.concatenate([dst, pad_dst]).reshape(NROW, B)
    xp = jnp.pad(x, ((0, NPAD - N), (0, 0)))
    zero_agg = jnp.zeros((NP, D), jnp.float32)
    zero_cnt = jnp.zeros((NCNT,), jnp.float32)

    agg1, cnt = _sc_agg1(xp, src2d, dst2d, zero_agg, zero_cnt)
    cnt4 = cnt.reshape(NW, NR, 1, B)
    h = _tc1(agg1, cnt4, xp, W_l1, b_l1, W_r1)
    agg2 = _sc_agg2(h, src2d, dst2d, zero_agg)
    out = _tc2(agg2, cnt4, h, W_l2, b_l2, W_r2, W_lin, b_lin)
    return out[:N]


# speedup vs baseline: 2.6920x; 2.6920x over previous
import jax
import jax.numpy as jnp
from jax import lax
from jax.experimental import pallas as pl
from jax.experimental.pallas import tpu as pltpu
from jax.experimental.pallas import tpu_sc as plsc

N = 10000
E = 320000
D = 128
H = 128
O = 64

NC = 2
NS = 16
NW = NC * NS
B = 128
CH = 8

EP = ((E + NW * B * CH - 1) // (NW * B * CH)) * (NW * B * CH)
NROW = EP // B
C0 = 10
C1 = 10
ROWS0 = C0 * CH
ROWS1 = C1 * CH

NP = 10112
NPT = NP // NS
NPAD = NP
NR = 80
NCNT = NR * B

_mesh = plsc.VectorSubcoreMesh(
    core_axis_name="c", subcore_axis_name="s", num_cores=NC, num_subcores=NS
)


def _agg_pipeline(feat, src2d, dst2d, agg_sh, c, s,
                  idx_s, idx_d, rows0, rows1, gs0, gs1, ss0, ss1,
                  cnt_v=None):
    bufs = (rows0, rows1)
    gsems = (gs0, gs1)
    ssems = (ss0, ss1)
    ones16 = jnp.ones((16,), jnp.float32)
    base = jnp.where(c == 0, s * ROWS0, NS * ROWS0 + s * ROWS1)
    nch = jnp.where(c == 0, C0, C1)

    def step(j, carry):
        row0 = base + j * CH
        pltpu.sync_copy(src2d.at[pl.ds(row0, CH)], idx_s)
        pltpu.sync_copy(dst2d.at[pl.ds(row0, CH)], idx_d)
        g = {}
        sc = {}
        g[0] = pltpu.async_copy(feat.at[idx_s.at[0]], bufs[0], gsems[0])
        for k in range(CH):
            if k + 1 < CH:
                if k >= 1:
                    sc[k - 1].wait()
                g[k + 1] = pltpu.async_copy(
                    feat.at[idx_s.at[k + 1]], bufs[(k + 1) % 2],
                    gsems[(k + 1) % 2])
            if cnt_v is not None:
                for gi in range(B // 16):
                    idxv = idx_d[k, pl.ds(gi * 16, 16)]
                    plsc.addupdate_scatter(cnt_v, [idxv], ones16)
            g[k].wait()
            sc[k] = pltpu.async_copy(bufs[k % 2], agg_sh.at[idx_d.at[k]],
                                     ssems[k % 2], add=True)
        sc[CH - 2].wait()
        sc[CH - 1].wait()
        return carry

    lax.fori_loop(0, nch, step, 0)


def _sc_agg1_body(feat, src2d, dst2d, zero_agg, zero_cnt,
                  agg_out, cnt_out,
                  idx_s, idx_d, rows0, rows1, cnt_v, agg_sh,
                  gs0, gs1, ss0, ss1):
    c = lax.axis_index("c")
    s = lax.axis_index("s")
    tid = c * NS + s
    r0 = s * NPT
    pltpu.sync_copy(zero_agg.at[pl.ds(r0, NPT)], agg_sh.at[pl.ds(r0, NPT)])
    pltpu.sync_copy(zero_cnt, cnt_v)
    plsc.subcore_barrier()
    _agg_pipeline(feat, src2d, dst2d, agg_sh, c, s,
                  idx_s, idx_d, rows0, rows1, gs0, gs1, ss0, ss1, cnt_v)
    plsc.subcore_barrier()
    pltpu.sync_copy(agg_sh.at[pl.ds(r0, NPT)], agg_out.at[c, pl.ds(r0, NPT)])
    pltpu.sync_copy(cnt_v, cnt_out.at[pl.ds(tid * NCNT, NCNT)])


def _sc_agg2_body(feat, src2d, dst2d, zero_agg,
                  agg_out,
                  idx_s, idx_d, rows0, rows1, agg_sh,
                  gs0, gs1, ss0, ss1):
    c = lax.axis_index("c")
    s = lax.axis_index("s")
    tid = c * NS + s
    r0 = s * NPT
    pltpu.sync_copy(zero_agg.at[pl.ds(r0, NPT)], agg_sh.at[pl.ds(r0, NPT)])
    plsc.subcore_barrier()
    _agg_pipeline(feat, src2d, dst2d, agg_sh, c, s,
                  idx_s, idx_d, rows0, rows1, gs0, gs1, ss0, ss1)
    plsc.subcore_barrier()
    pltpu.sync_copy(agg_sh.at[pl.ds(r0, NPT)], agg_out.at[c, pl.ds(r0, NPT)])


_sc_agg1 = pl.kernel(
    _sc_agg1_body,
    out_type=[
        jax.ShapeDtypeStruct((NC, NP, D), jnp.float32),
        jax.ShapeDtypeStruct((NW * NCNT,), jnp.float32),
    ],
    mesh=_mesh,
    compiler_params=pltpu.CompilerParams(needs_layout_passes=False),
    scratch_types=[
        pltpu.VMEM((CH, B), jnp.int32),
        pltpu.VMEM((CH, B), jnp.int32),
        pltpu.VMEM((B, D), jnp.float32),
        pltpu.VMEM((B, D), jnp.float32),
        pltpu.VMEM((NCNT,), jnp.float32),
        pltpu.VMEM_SHARED((NP, D), jnp.float32),
        pltpu.SemaphoreType.DMA,
        pltpu.SemaphoreType.DMA,
        pltpu.SemaphoreType.DMA,
        pltpu.SemaphoreType.DMA,
    ],
)

_sc_agg2 = pl.kernel(
    _sc_agg2_body,
    out_type=jax.ShapeDtypeStruct((NC, NP, H), jnp.float32),
    mesh=_mesh,
    scratch_types=[
        pltpu.VMEM((CH, B), jnp.int32),
        pltpu.VMEM((CH, B), jnp.int32),
        pltpu.VMEM((B, H), jnp.float32),
        pltpu.VMEM((B, H), jnp.float32),
        pltpu.VMEM_SHARED((NP, H), jnp.float32),
        pltpu.SemaphoreType.DMA,
        pltpu.SemaphoreType.DMA,
        pltpu.SemaphoreType.DMA,
        pltpu.SemaphoreType.DMA,
    ],
)

_BR = 128


PAD_E = EP - E


def _cnt_col(cntp):
    cnt = jnp.sum(cntp[:, 0, 0, :], axis=0)
    rowid = pl.program_id(0) * B + lax.iota(jnp.int32, B)
    cnt = cnt - jnp.where(rowid < PAD_E, 1.0, 0.0)
    return jnp.maximum(cnt, 1.0).reshape(B, 1)


def _tc1_body(aggp, cntp, x, wl, bl, wr, out):
    agg = aggp[0] + aggp[1]
    mean = agg / _cnt_col(cntp)
    r = lax.dot_general(mean, wl[...], (((1,), (1,)), ((), ())),
                        preferred_element_type=jnp.float32)
    r = r + bl[...]
    r = r + lax.dot_general(x[...], wr[...], (((1,), (1,)), ((), ())),
                            preferred_element_type=jnp.float32)
    rowid = pl.program_id(0) * B + lax.broadcasted_iota(jnp.int32, (B, 1), 0)[:, 0]
    keep = (rowid < N).reshape(B, 1)
    out[...] = jnp.where(keep, jnp.maximum(r, 0.0), 0.0)


def _tc2_body(aggp, cntp, h, wl, bl, wr, wlin, blin, out):
    agg = aggp[0] + aggp[1]
    mean = agg / _cnt_col(cntp)
    r = lax.dot_general(mean, wl[...], (((1,), (1,)), ((), ())),
                        preferred_element_type=jnp.float32)
    r = r + bl[...]
    r = r + lax.dot_general(h[...], wr[...], (((1,), (1,)), ((), ())),
                            preferred_element_type=jnp.float32)
    logits = lax.dot_general(r, wlin[...], (((1,), (1,)), ((), ())),
                             preferred_element_type=jnp.float32)
    logits = logits + blin[...]
    m = jnp.max(logits, axis=1, keepdims=True)
    lse = jnp.log(jnp.sum(jnp.exp(logits - m), axis=1, keepdims=True)) + m
    out[...] = logits - lse


def _tc1(agg_parts, cnt_parts, xp, W_l1, b_l1, W_r1):
    return pl.pallas_call(
        _tc1_body,
        grid=(NPAD // _BR,),
        in_specs=[
            pl.BlockSpec((NC, _BR, D), lambda i: (0, i, 0)),
            pl.BlockSpec((NW, 1, 1, B), lambda i: (0, i, 0, 0)),
            pl.BlockSpec((_BR, D), lambda i: (i, 0)),
            pl.BlockSpec((H, D), lambda i: (0, 0)),
            pl.BlockSpec((1, H), lambda i: (0, 0)),
            pl.BlockSpec((H, D), lambda i: (0, 0)),
        ],
        out_specs=pl.BlockSpec((_BR, H), lambda i: (i, 0)),
        out_shape=jax.ShapeDtypeStruct((NPAD, H), jnp.float32),
    )(agg_parts, cnt_parts, xp, W_l1, b_l1.reshape(1, H), W_r1)


def _tc2(agg_parts, cnt_parts, h, W_l2, b_l2, W_r2, W_lin, b_lin):
    return pl.pallas_call(
        _tc2_body,
        grid=(NPAD // _BR,),
        in_specs=[
            pl.BlockSpec((NC, _BR, H), lambda i: (0, i, 0)),
            pl.BlockSpec((NW, 1, 1, B), lambda i: (0, i, 0, 0)),
            pl.BlockSpec((_BR, H), lambda i: (i, 0)),
            pl.BlockSpec((H, H), lambda i: (0, 0)),
            pl.BlockSpec((1, H), lambda i: (0, 0)),
            pl.BlockSpec((H, H), lambda i: (0, 0)),
            pl.BlockSpec((O, H), lambda i: (0, 0)),
            pl.BlockSpec((1, O), lambda i: (0, 0)),
        ],
        out_specs=pl.BlockSpec((_BR, O), lambda i: (i, 0)),
        out_shape=jax.ShapeDtypeStruct((NPAD, O), jnp.float32),
    )(agg_parts, cnt_parts, h, W_l2, b_l2.reshape(1, H), W_r2,
      W_lin, b_lin.reshape(1, O))


def kernel(x, edge_index, W_l1, b_l1, W_r1, W_l2, b_l2, W_r2, W_lin, b_lin):
    src = edge_index[0]
    dst = edge_index[1]
    pad = EP - E
    pad_src = N + (jnp.arange(pad, dtype=jnp.int32) % (NP - N))
    src2d = jnp.concatenate([src, pad_src]).reshape(NROW, B)
    pad_dst = jnp.arange(pad, dtype=jnp.int32)
    dst2d = jnp.concatenate([dst, pad_dst]).reshape(NROW, B)
    src2d = src2d.reshape(NROW // NW, NW, B).transpose(1, 0, 2).reshape(NROW, B)
    dst2d = dst2d.reshape(NROW // NW, NW, B).transpose(1, 0, 2).reshape(NROW, B)
    xp = jnp.pad(x, ((0, NPAD - N), (0, 0)))
    zero_agg = jnp.zeros((NP, D), jnp.float32)
    zero_cnt = jnp.zeros((NCNT,), jnp.float32)

    agg1, cnt = _sc_agg1(xp, src2d, dst2d, zero_agg, zero_cnt)
    cnt4 = cnt.reshape(NW, NR, 1, B)
    h = _tc1(agg1, cnt4, xp, W_l1, b_l1, W_r1)
    agg2 = _sc_agg2(h, src2d, dst2d, zero_agg)
    out = _tc2(agg2, cnt4, h, W_l2, b_l2, W_r2, W_lin, b_lin)
    return out[:N]


# speedup vs baseline: 3.3757x; 1.2540x over previous
import jax
import jax.numpy as jnp
from jax import lax
from jax.experimental import pallas as pl
from jax.experimental.pallas import tpu as pltpu
from jax.experimental.pallas import tpu_sc as plsc

N = 10000
E = 320000
D = 128
H = 128
O = 64

NC = 2
NS = 16
NW = NC * NS
B = 128
CH = 8

EP = ((E + NW * B * CH - 1) // (NW * B * CH)) * (NW * B * CH)
NROW = EP // B
C0 = 10
C1 = 10
ROWS0 = C0 * CH
ROWS1 = C1 * CH

NP = 10240
NPT = NP // NS
NPAD = NP
NR = 80
NCNT = NR * B

_mesh = plsc.VectorSubcoreMesh(
    core_axis_name="c", subcore_axis_name="s", num_cores=NC, num_subcores=NS
)


def _agg_pipeline(feat, src2d, dst2d, agg_sh, c, s,
                  idx_s, idx_d, rows0, rows1, gs0, gs1, ss0, ss1,
                  cnt_v=None):
    bufs = (rows0, rows1)
    gsems = (gs0, gs1)
    ssems = (ss0, ss1)
    ones16 = jnp.ones((16,), jnp.float32)
    base = jnp.where(c == 0, s * ROWS0, NS * ROWS0 + s * ROWS1)
    nch = jnp.where(c == 0, C0, C1)

    def step(j, carry):
        row0 = base + j * CH
        pltpu.sync_copy(src2d.at[pl.ds(row0, CH)], idx_s)
        pltpu.sync_copy(dst2d.at[pl.ds(row0, CH)], idx_d)
        g = {}
        sc = {}
        g[0] = pltpu.async_copy(feat.at[idx_s.at[0]], bufs[0], gsems[0])
        for k in range(CH):
            if k + 1 < CH:
                if k >= 1:
                    sc[k - 1].wait()
                g[k + 1] = pltpu.async_copy(
                    feat.at[idx_s.at[k + 1]], bufs[(k + 1) % 2],
                    gsems[(k + 1) % 2])
            if cnt_v is not None:
                for gi in range(B // 16):
                    idxv = idx_d[k, pl.ds(gi * 16, 16)]
                    plsc.addupdate_scatter(cnt_v, [idxv], ones16)
            g[k].wait()
            sc[k] = pltpu.async_copy(bufs[k % 2], agg_sh.at[idx_d.at[k]],
                                     ssems[k % 2], add=True)
        sc[CH - 2].wait()
        sc[CH - 1].wait()
        return carry

    lax.fori_loop(0, nch, step, 0)


def _sc_agg1_body(feat, src2d, dst2d, zero_agg, zero_cnt,
                  agg_out, cnt_out,
                  idx_s, idx_d, rows0, rows1, cnt_v, agg_sh,
                  gs0, gs1, ss0, ss1):
    c = lax.axis_index("c")
    s = lax.axis_index("s")
    tid = c * NS + s
    r0 = s * NPT
    pltpu.sync_copy(zero_agg.at[pl.ds(r0, NPT)], agg_sh.at[pl.ds(r0, NPT)])
    pltpu.sync_copy(zero_cnt, cnt_v)
    plsc.subcore_barrier()
    _agg_pipeline(feat, src2d, dst2d, agg_sh, c, s,
                  idx_s, idx_d, rows0, rows1, gs0, gs1, ss0, ss1, cnt_v)
    plsc.subcore_barrier()
    pltpu.sync_copy(agg_sh.at[pl.ds(r0, NPT)], agg_out.at[c, pl.ds(r0, NPT)])
    pltpu.sync_copy(cnt_v, cnt_out.at[pl.ds(tid * NCNT, NCNT)])


def _sc_agg2_body(feat, src2d, dst2d, zero_agg,
                  agg_out,
                  idx_s, idx_d, rows0, rows1, agg_sh,
                  gs0, gs1, ss0, ss1):
    c = lax.axis_index("c")
    s = lax.axis_index("s")
    tid = c * NS + s
    r0 = s * NPT
    pltpu.sync_copy(zero_agg.at[pl.ds(r0, NPT)], agg_sh.at[pl.ds(r0, NPT)])
    plsc.subcore_barrier()
    _agg_pipeline(feat, src2d, dst2d, agg_sh, c, s,
                  idx_s, idx_d, rows0, rows1, gs0, gs1, ss0, ss1)
    plsc.subcore_barrier()
    pltpu.sync_copy(agg_sh.at[pl.ds(r0, NPT)], agg_out.at[c, pl.ds(r0, NPT)])


_sc_agg1 = pl.kernel(
    _sc_agg1_body,
    out_type=[
        jax.ShapeDtypeStruct((NC, NP, D), jnp.float32),
        jax.ShapeDtypeStruct((NW * NCNT,), jnp.float32),
    ],
    mesh=_mesh,
    compiler_params=pltpu.CompilerParams(needs_layout_passes=False),
    scratch_types=[
        pltpu.VMEM((CH, B), jnp.int32),
        pltpu.VMEM((CH, B), jnp.int32),
        pltpu.VMEM((B, D), jnp.float32),
        pltpu.VMEM((B, D), jnp.float32),
        pltpu.VMEM((NCNT,), jnp.float32),
        pltpu.VMEM_SHARED((NP, D), jnp.float32),
        pltpu.SemaphoreType.DMA,
        pltpu.SemaphoreType.DMA,
        pltpu.SemaphoreType.DMA,
        pltpu.SemaphoreType.DMA,
    ],
)

_sc_agg2 = pl.kernel(
    _sc_agg2_body,
    out_type=jax.ShapeDtypeStruct((NC, NP, H), jnp.float32),
    mesh=_mesh,
    scratch_types=[
        pltpu.VMEM((CH, B), jnp.int32),
        pltpu.VMEM((CH, B), jnp.int32),
        pltpu.VMEM((B, H), jnp.float32),
        pltpu.VMEM((B, H), jnp.float32),
        pltpu.VMEM_SHARED((NP, H), jnp.float32),
        pltpu.SemaphoreType.DMA,
        pltpu.SemaphoreType.DMA,
        pltpu.SemaphoreType.DMA,
        pltpu.SemaphoreType.DMA,
    ],
)

_BR = 1280
_KR = _BR // B


PAD_E = EP - E


def _cnt_col(cntp):
    cnt = jnp.sum(cntp[:, :, 0, :], axis=0)
    base = pl.program_id(0) * _BR
    cols = []
    for r in range(_KR):
        rowid = base + r * B + lax.iota(jnp.int32, B)
        c = cnt[r] - jnp.where(rowid < PAD_E, 1.0, 0.0)
        cols.append(jnp.maximum(c, 1.0).reshape(B, 1))
    return jnp.concatenate(cols, axis=0)


def _tc1_body(aggp, cntp, x, wl, bl, wr, out):
    agg = aggp[0] + aggp[1]
    mean = agg / _cnt_col(cntp)
    r = lax.dot_general(mean, wl[...], (((1,), (1,)), ((), ())),
                        preferred_element_type=jnp.float32)
    r = r + bl[...]
    r = r + lax.dot_general(x[...], wr[...], (((1,), (1,)), ((), ())),
                            preferred_element_type=jnp.float32)
    rowid = pl.program_id(0) * _BR + lax.broadcasted_iota(jnp.int32, (_BR, 1), 0)
    out[...] = jnp.where(rowid < N, jnp.maximum(r, 0.0), 0.0)


def _tc2_body(aggp, cntp, h, wl, bl, wr, wlin, blin, out):
    agg = aggp[0] + aggp[1]
    mean = agg / _cnt_col(cntp)
    r = lax.dot_general(mean, wl[...], (((1,), (1,)), ((), ())),
                        preferred_element_type=jnp.float32)
    r = r + bl[...]
    r = r + lax.dot_general(h[...], wr[...], (((1,), (1,)), ((), ())),
                            preferred_element_type=jnp.float32)
    logits = lax.dot_general(r, wlin[...], (((1,), (1,)), ((), ())),
                             preferred_element_type=jnp.float32)
    logits = logits + blin[...]
    m = jnp.max(logits, axis=1, keepdims=True)
    lse = jnp.log(jnp.sum(jnp.exp(logits - m), axis=1, keepdims=True)) + m
    out[...] = logits - lse


def _tc1(agg_parts, cnt_parts, xp, W_l1, b_l1, W_r1):
    return pl.pallas_call(
        _tc1_body,
        grid=(NPAD // _BR,),
        in_specs=[
            pl.BlockSpec((NC, _BR, D), lambda i: (0, i, 0)),
            pl.BlockSpec((NW, _KR, 1, B), lambda i: (0, i, 0, 0)),
            pl.BlockSpec((_BR, D), lambda i: (i, 0)),
            pl.BlockSpec((H, D), lambda i: (0, 0)),
            pl.BlockSpec((1, H), lambda i: (0, 0)),
            pl.BlockSpec((H, D), lambda i: (0, 0)),
        ],
        out_specs=pl.BlockSpec((_BR, H), lambda i: (i, 0)),
        out_shape=jax.ShapeDtypeStruct((NPAD, H), jnp.float32),
    )(agg_parts, cnt_parts, xp, W_l1, b_l1.reshape(1, H), W_r1)


def _tc2(agg_parts, cnt_parts, h, W_l2, b_l2, W_r2, W_lin, b_lin):
    return pl.pallas_call(
        _tc2_body,
        grid=(NPAD // _BR,),
        in_specs=[
            pl.BlockSpec((NC, _BR, H), lambda i: (0, i, 0)),
            pl.BlockSpec((NW, _KR, 1, B), lambda i: (0, i, 0, 0)),
            pl.BlockSpec((_BR, H), lambda i: (i, 0)),
            pl.BlockSpec((H, H), lambda i: (0, 0)),
            pl.BlockSpec((1, H), lambda i: (0, 0)),
            pl.BlockSpec((H, H), lambda i: (0, 0)),
            pl.BlockSpec((O, H), lambda i: (0, 0)),
            pl.BlockSpec((1, O), lambda i: (0, 0)),
        ],
        out_specs=pl.BlockSpec((_BR, O), lambda i: (i, 0)),
        out_shape=jax.ShapeDtypeStruct((NPAD, O), jnp.float32),
    )(agg_parts, cnt_parts, h, W_l2, b_l2.reshape(1, H), W_r2,
      W_lin, b_lin.reshape(1, O))


def kernel(x, edge_index, W_l1, b_l1, W_r1, W_l2, b_l2, W_r2, W_lin, b_lin):
    src = edge_index[0]
    dst = edge_index[1]
    pad = EP - E
    pad_src = N + (jnp.arange(pad, dtype=jnp.int32) % (NP - N))
    src2d = jnp.concatenate([src, pad_src]).reshape(NROW, B)
    pad_dst = jnp.arange(pad, dtype=jnp.int32)
    dst2d = jnp.concatenate([dst, pad_dst]).reshape(NROW, B)
    src2d = src2d.reshape(NROW // NW, NW, B).transpose(1, 0, 2).reshape(NROW, B)
    dst2d = dst2d.reshape(NROW // NW, NW, B).transpose(1, 0, 2).reshape(NROW, B)
    xp = jnp.pad(x, ((0, NPAD - N), (0, 0)))
    zero_agg = jnp.zeros((NP, D), jnp.float32)
    zero_cnt = jnp.zeros((NCNT,), jnp.float32)

    agg1, cnt = _sc_agg1(xp, src2d, dst2d, zero_agg, zero_cnt)
    cnt4 = cnt.reshape(NW, NR, 1, B)
    h = _tc1(agg1, cnt4, xp, W_l1, b_l1, W_r1)
    agg2 = _sc_agg2(h, src2d, dst2d, zero_agg)
    out = _tc2(agg2, cnt4, h, W_l2, b_l2, W_r2, W_lin, b_lin)
    return out[:N]
